# Initial kernel scaffold; baseline (speedup 1.0000x reference)
import jax
import jax.numpy as jnp
from jax.experimental import pallas as pl


def kernel(x, edge_index, W_in, b_in, W_convs, W_out, b_out):
    raise NotImplementedError("write your pallas kernel here")


# speedup vs baseline: 13.6372x; 13.6372x over previous
import functools

import jax
import jax.numpy as jnp
from jax import lax
from jax.experimental import pallas as pl
from jax.experimental.pallas import tpu as pltpu
from jax.experimental.pallas import tpu_sc as plsc

N = 10000
E = 320000
D = 128
ALPHA = 0.1

NC = 2
NS = 16
NW = NC * NS
EPT = E // NW
CHUNK = 80
NCHUNK = EPT // CHUNK
RPT = N // NS
ROWS0 = 640
ROWSR = (N - ROWS0) // (NS - 1)
RB = 1000
GRID = N // RB

def _deg_body(dst_hbm, out_hbm, dstbuf, degbuf):
    c = lax.axis_index("c")
    s = lax.axis_index("s")
    wid = c * NS + s
    pltpu.sync_copy(dst_hbm.at[pl.ds(wid * EPT, EPT)], dstbuf)

    def zero(i, carry):
        degbuf[pl.ds(i * 16, 16)] = jnp.zeros((16,), jnp.float32)
        return carry

    lax.fori_loop(0, N // 16, zero, 0)
    ones = jnp.ones((16,), jnp.float32)

    def count(i, carry):
        dv = dstbuf[pl.ds(i * 16, 16)]
        plsc.addupdate_scatter(degbuf, [dv], ones)
        return carry

    lax.fori_loop(0, EPT // 16, count, 0)
    pltpu.sync_copy(degbuf, out_hbm.at[wid])


def _edge_body(t_hbm, src_hbm, dst_hbm, out_hbm, sidx, didx, rows, acc, gsem):
    c = lax.axis_index("c")
    s = lax.axis_index("s")

    def _seed_writeback(copy_fn):
        @pl.when(s == 0)
        def _():
            copy_fn(0, ROWS0)

        @pl.when(s > 0)
        def _():
            copy_fn(ROWS0 + (s - 1) * ROWSR, ROWSR)

    _seed_writeback(lambda off, nrows: pltpu.sync_copy(
        t_hbm.at[pl.ds(off, nrows)], acc.at[pl.ds(off, nrows)]))
    plsc.subcore_barrier()

    base_e = (c * NS + s) * EPT

    def body(i, carry):
        off = base_e + i * CHUNK
        pltpu.sync_copy(src_hbm.at[pl.ds(off, CHUNK)], sidx)
        pltpu.sync_copy(dst_hbm.at[pl.ds(off, CHUNK)], didx)
        pltpu.async_copy(t_hbm.at[sidx], rows, gsem).wait()
        pltpu.sync_copy(rows, acc.at[didx], add=True)
        return carry

    lax.fori_loop(0, NCHUNK, body, 0)
    plsc.subcore_barrier()
    _seed_writeback(lambda off, nrows: pltpu.sync_copy(
        acc.at[pl.ds(off, nrows)], out_hbm.at[pl.ds(c * N + off, nrows)]))


@functools.lru_cache(maxsize=None)
def _sc_calls():
    mesh = plsc.VectorSubcoreMesh(
        core_axis_name="c", subcore_axis_name="s",
        num_cores=NC, num_subcores=NS)
    deg_call = pl.kernel(
        _deg_body,
        out_type=jax.ShapeDtypeStruct((NW, N), jnp.float32),
        mesh=mesh,
        scratch_types=[
            pltpu.VMEM((EPT,), jnp.int32),
            pltpu.VMEM((N,), jnp.float32),
        ],
        compiler_params=pltpu.CompilerParams(needs_layout_passes=False),
    )
    edge_call = pl.kernel(
        _edge_body,
        out_type=jax.ShapeDtypeStruct((2 * N, D), jnp.float32),
        mesh=mesh,
        scratch_types=[
            pltpu.VMEM((CHUNK,), jnp.int32),
            pltpu.VMEM((CHUNK,), jnp.int32),
            pltpu.VMEM((CHUNK, D), jnp.float32),
            pltpu.VMEM_SHARED((N, D), jnp.float32),
            pltpu.SemaphoreType.DMA,
        ],
    )
    return deg_call, edge_call


def _dinv_body(parts_ref, out_ref):
    deg = jnp.sum(parts_ref[...], axis=0, keepdims=True) + 1.0
    out_ref[...] = lax.rsqrt(deg)


def _proj_body(x_ref, w_ref, b_ref, dinv_ref, h_ref, t_ref):
    h = jnp.dot(x_ref[...], w_ref[...],
                preferred_element_type=jnp.float32,
                precision=lax.Precision.HIGHEST)
    h = jnp.maximum(h + b_ref[...], 0.0)
    h_ref[...] = h
    t_ref[...] = h * dinv_ref[...]


def _layer1_body(p0_ref, p1_ref, t_ref, x0_ref, dinv_ref, w_ref, o_ref):
    dinv = dinv_ref[...]
    s = p0_ref[...] + p1_ref[...] - t_ref[...]
    hm = (1.0 - ALPHA) * (dinv * s) + ALPHA * x0_ref[...]
    cur = jnp.maximum(
        jnp.dot(hm, w_ref[...], preferred_element_type=jnp.float32,
                precision=lax.Precision.HIGHEST), 0.0)
    o_ref[...] = dinv * cur


def _layer2_body(p0_ref, p1_ref, t_ref, x0_ref, dinv_ref, w_ref, wo_ref,
                 bo_ref, o_ref):
    dinv = dinv_ref[...]
    s = p0_ref[...] + p1_ref[...] - t_ref[...]
    hm = (1.0 - ALPHA) * (dinv * s) + ALPHA * x0_ref[...]
    cur = jnp.maximum(
        jnp.dot(hm, w_ref[...], preferred_element_type=jnp.float32,
                precision=lax.Precision.HIGHEST), 0.0)
    o_ref[...] = jnp.dot(cur, wo_ref[...], preferred_element_type=jnp.float32,
                         precision=lax.Precision.HIGHEST) + bo_ref[...]


def _row_spec(i_map=lambda i: (i, 0)):
    return pl.BlockSpec((RB, D), i_map)


_full_w = pl.BlockSpec((D, D), lambda i: (0, 0))
_full_b = pl.BlockSpec((1, D), lambda i: (0, 0))
_dinv_spec = pl.BlockSpec((RB, 1), lambda i: (i, 0))

_proj_call = pl.pallas_call(
    _proj_body,
    grid=(GRID,),
    in_specs=[_row_spec(), _full_w, _full_b, _dinv_spec],
    out_specs=[_row_spec(), _row_spec()],
    out_shape=[jax.ShapeDtypeStruct((N, D), jnp.float32),
               jax.ShapeDtypeStruct((N, D), jnp.float32)],
)

_layer1_call = pl.pallas_call(
    _layer1_body,
    grid=(GRID,),
    in_specs=[_row_spec(), _row_spec(lambda i: (GRID + i, 0)), _row_spec(),
              _row_spec(), _dinv_spec, _full_w],
    out_specs=_row_spec(),
    out_shape=jax.ShapeDtypeStruct((N, D), jnp.float32),
)

_layer2_call = pl.pallas_call(
    _layer2_body,
    grid=(GRID,),
    in_specs=[_row_spec(), _row_spec(lambda i: (GRID + i, 0)), _row_spec(),
              _row_spec(), _dinv_spec, _full_w, _full_w, _full_b],
    out_specs=_row_spec(),
    out_shape=jax.ShapeDtypeStruct((N, D), jnp.float32),
)

_dinv_call = pl.pallas_call(
    _dinv_body,
    out_shape=jax.ShapeDtypeStruct((1, N), jnp.float32),
)


def kernel(x, edge_index, W_in, b_in, W_convs, W_out, b_out):
    src = edge_index[0].astype(jnp.int32)
    dst = edge_index[1].astype(jnp.int32)
    _deg_call, _edge_call = _sc_calls()

    deg_parts = _deg_call(dst)
    dinv2d = _dinv_call(deg_parts)
    dinv_col = dinv2d.reshape(N, 1)

    h, t0 = _proj_call(x, W_in, b_in.reshape(1, D), dinv_col)

    p1 = _edge_call(t0, src, dst)
    t1 = _layer1_call(p1, p1, t0, h, dinv_col, W_convs[0])

    p2 = _edge_call(t1, src, dst)
    y = _layer2_call(p2, p2, t1, h, dinv_col, W_convs[1], W_out,
                     b_out.reshape(1, D))
    return y


# speedup vs baseline: 18.5976x; 1.3637x over previous
import functools

import jax
import jax.numpy as jnp
from jax import lax
from jax.experimental import pallas as pl
from jax.experimental.pallas import tpu as pltpu
from jax.experimental.pallas import tpu_sc as plsc

N = 10000
E = 320000
D = 128
ALPHA = 0.1

NC = 2
NS = 16
NW = NC * NS
EPT = E // NW
CHUNK = 80
NCHUNK = EPT // CHUNK
RPT = N // NS
ROWS0 = 640
ROWSR = (N - ROWS0) // (NS - 1)
RB = 1000
GRID = N // RB

def _deg_body(dst_hbm, out_hbm, dstbuf, degbuf):
    c = lax.axis_index("c")
    s = lax.axis_index("s")
    wid = c * NS + s
    pltpu.sync_copy(dst_hbm.at[pl.ds(wid * EPT, EPT)], dstbuf)

    def zero(i, carry):
        degbuf[pl.ds(i * 16, 16)] = jnp.zeros((16,), jnp.float32)
        return carry

    lax.fori_loop(0, N // 16, zero, 0)
    ones = jnp.ones((16,), jnp.float32)

    def count(i, carry):
        dv = dstbuf[pl.ds(i * 16, 16)]
        plsc.addupdate_scatter(degbuf, [dv], ones)
        return carry

    lax.fori_loop(0, EPT // 16, count, 0)
    pltpu.sync_copy(degbuf, out_hbm.at[wid])


def _edge_body(t_hbm, src_hbm, dst_hbm, out_hbm, sidx, didx, rows, acc, gsem):
    c = lax.axis_index("c")
    s = lax.axis_index("s")
    wid = c * NS + s

    def _seed_writeback(copy_fn):
        @pl.when(s == 0)
        def _():
            copy_fn(0, ROWS0)

        @pl.when(s > 0)
        def _():
            copy_fn(ROWS0 + (s - 1) * ROWSR, ROWSR)

    _seed_writeback(lambda off, nrows: pltpu.sync_copy(
        t_hbm.at[pl.ds(off, nrows)], acc.at[pl.ds(off, nrows)]))
    pltpu.sync_copy(dst_hbm.at[wid], didx)
    plsc.subcore_barrier()

    def _gather(i, slot, sem):
        pltpu.sync_copy(src_hbm.at[wid, i], sidx[slot])
        pltpu.async_copy(t_hbm.at[sidx[slot]], rows[slot], sem)

    def _gwait(slot, sem):
        pltpu.make_async_copy(t_hbm.at[sidx[slot]], rows[slot], sem).wait()

    _gather(0, 0, gsem[0])

    def pair(j, carry):
        i0 = 2 * j
        _gwait(0, gsem[0])
        _gather(i0 + 1, 1, gsem[1])
        pltpu.sync_copy(rows[0], acc.at[didx.at[i0]], add=True)
        _gwait(1, gsem[1])
        _gather(i0 + 2, 0, gsem[0])
        pltpu.sync_copy(rows[1], acc.at[didx.at[i0 + 1]], add=True)
        return carry

    lax.fori_loop(0, (NCHUNK - 1) // 2, pair, 0)
    _gwait(0, gsem[0])
    pltpu.sync_copy(rows[0], acc.at[didx.at[NCHUNK - 1]], add=True)
    plsc.subcore_barrier()
    _seed_writeback(lambda off, nrows: pltpu.sync_copy(
        acc.at[pl.ds(off, nrows)], out_hbm.at[pl.ds(c * N + off, nrows)]))


@functools.lru_cache(maxsize=None)
def _sc_calls():
    mesh = plsc.VectorSubcoreMesh(
        core_axis_name="c", subcore_axis_name="s",
        num_cores=NC, num_subcores=NS)
    deg_call = pl.kernel(
        _deg_body,
        out_type=jax.ShapeDtypeStruct((NW, N), jnp.float32),
        mesh=mesh,
        scratch_types=[
            pltpu.VMEM((EPT,), jnp.int32),
            pltpu.VMEM((N,), jnp.float32),
        ],
        compiler_params=pltpu.CompilerParams(needs_layout_passes=False),
    )
    edge_call = pl.kernel(
        _edge_body,
        out_type=jax.ShapeDtypeStruct((2 * N, D), jnp.float32),
        mesh=mesh,
        scratch_types=[
            [pltpu.VMEM((CHUNK,), jnp.int32) for _ in range(2)],
            pltpu.VMEM((NCHUNK, CHUNK), jnp.int32),
            [pltpu.VMEM((CHUNK, D), jnp.float32) for _ in range(2)],
            pltpu.VMEM_SHARED((N, D), jnp.float32),
            [pltpu.SemaphoreType.DMA for _ in range(2)],
        ],
    )
    return deg_call, edge_call


def _dinv_body(parts_ref, out_ref):
    deg = jnp.sum(parts_ref[...], axis=0, keepdims=True) + 1.0
    out_ref[...] = lax.rsqrt(deg)


def _proj_body(x_ref, w_ref, b_ref, dinv_ref, h_ref, t_ref):
    h = jnp.dot(x_ref[...], w_ref[...],
                preferred_element_type=jnp.float32,
                precision=lax.Precision.HIGHEST)
    h = jnp.maximum(h + b_ref[...], 0.0)
    h_ref[...] = h
    t_ref[...] = h * dinv_ref[...]


def _layer1_body(p0_ref, p1_ref, t_ref, x0_ref, dinv_ref, w_ref, o_ref):
    dinv = dinv_ref[...]
    s = p0_ref[...] + p1_ref[...] - t_ref[...]
    hm = (1.0 - ALPHA) * (dinv * s) + ALPHA * x0_ref[...]
    cur = jnp.maximum(
        jnp.dot(hm, w_ref[...], preferred_element_type=jnp.float32,
                precision=lax.Precision.HIGHEST), 0.0)
    o_ref[...] = dinv * cur


def _layer2_body(p0_ref, p1_ref, t_ref, x0_ref, dinv_ref, w_ref, wo_ref,
                 bo_ref, o_ref):
    dinv = dinv_ref[...]
    s = p0_ref[...] + p1_ref[...] - t_ref[...]
    hm = (1.0 - ALPHA) * (dinv * s) + ALPHA * x0_ref[...]
    cur = jnp.maximum(
        jnp.dot(hm, w_ref[...], preferred_element_type=jnp.float32,
                precision=lax.Precision.HIGHEST), 0.0)
    o_ref[...] = jnp.dot(cur, wo_ref[...], preferred_element_type=jnp.float32,
                         precision=lax.Precision.HIGHEST) + bo_ref[...]


def _row_spec(i_map=lambda i: (i, 0)):
    return pl.BlockSpec((RB, D), i_map)


_full_w = pl.BlockSpec((D, D), lambda i: (0, 0))
_full_b = pl.BlockSpec((1, D), lambda i: (0, 0))
_dinv_spec = pl.BlockSpec((RB, 1), lambda i: (i, 0))

_proj_call = pl.pallas_call(
    _proj_body,
    grid=(GRID,),
    in_specs=[_row_spec(), _full_w, _full_b, _dinv_spec],
    out_specs=[_row_spec(), _row_spec()],
    out_shape=[jax.ShapeDtypeStruct((N, D), jnp.float32),
               jax.ShapeDtypeStruct((N, D), jnp.float32)],
)

_layer1_call = pl.pallas_call(
    _layer1_body,
    grid=(GRID,),
    in_specs=[_row_spec(), _row_spec(lambda i: (GRID + i, 0)), _row_spec(),
              _row_spec(), _dinv_spec, _full_w],
    out_specs=_row_spec(),
    out_shape=jax.ShapeDtypeStruct((N, D), jnp.float32),
)

_layer2_call = pl.pallas_call(
    _layer2_body,
    grid=(GRID,),
    in_specs=[_row_spec(), _row_spec(lambda i: (GRID + i, 0)), _row_spec(),
              _row_spec(), _dinv_spec, _full_w, _full_w, _full_b],
    out_specs=_row_spec(),
    out_shape=jax.ShapeDtypeStruct((N, D), jnp.float32),
)

_dinv_call = pl.pallas_call(
    _dinv_body,
    out_shape=jax.ShapeDtypeStruct((1, N), jnp.float32),
)


def kernel(x, edge_index, W_in, b_in, W_convs, W_out, b_out):
    src = edge_index[0].astype(jnp.int32)
    dst = edge_index[1].astype(jnp.int32)
    _deg_call, _edge_call = _sc_calls()

    deg_parts = _deg_call(dst)
    dinv2d = _dinv_call(deg_parts)
    dinv_col = dinv2d.reshape(N, 1)

    h, t0 = _proj_call(x, W_in, b_in.reshape(1, D), dinv_col)

    src3d = src.reshape(NW, NCHUNK, CHUNK)
    dst3d = dst.reshape(NW, NCHUNK, CHUNK)
    p1 = _edge_call(t0, src3d, dst3d)
    t1 = _layer1_call(p1, p1, t0, h, dinv_col, W_convs[0])

    p2 = _edge_call(t1, src3d, dst3d)
    y = _layer2_call(p2, p2, t1, h, dinv_col, W_convs[1], W_out,
                     b_out.reshape(1, D))
    return y


# speedup vs baseline: 18.9180x; 1.0172x over previous
import functools

import jax
import jax.numpy as jnp
from jax import lax
from jax.experimental import pallas as pl
from jax.experimental.pallas import tpu as pltpu
from jax.experimental.pallas import tpu_sc as plsc

N = 10000
E = 320000
D = 128
ALPHA = 0.1

NC = 2
NS = 16
NW = NC * NS
EPT = E // NW
CHUNK = 80
NCHUNK = EPT // CHUNK
RPT = N // NS
ROWS0 = 640
ROWSR = (N - ROWS0) // (NS - 1)
RB = 1000
GRID = N // RB

def _deg_body(dst_hbm, out_hbm, dstbuf, degbuf):
    c = lax.axis_index("c")
    s = lax.axis_index("s")
    wid = c * NS + s
    pltpu.sync_copy(dst_hbm.at[pl.ds(wid * EPT, EPT)], dstbuf)

    def zero(i, carry):
        degbuf[pl.ds(i * 16, 16)] = jnp.zeros((16,), jnp.float32)
        return carry

    lax.fori_loop(0, N // 16, zero, 0)
    ones = jnp.ones((16,), jnp.float32)

    def count(i, carry):
        dv = dstbuf[pl.ds(i * 16, 16)]
        plsc.addupdate_scatter(degbuf, [dv], ones)
        return carry

    lax.fori_loop(0, EPT // 16, count, 0)
    pltpu.sync_copy(degbuf, out_hbm.at[wid])


def _edge_body(t_hbm, src_hbm, dst_hbm, out_hbm, sidx, didx, rows, acc, gsem,
               ssem):
    c = lax.axis_index("c")
    s = lax.axis_index("s")
    wid = c * NS + s

    def _seed_writeback(copy_fn):
        @pl.when(s == 0)
        def _():
            copy_fn(0, ROWS0)

        @pl.when(s > 0)
        def _():
            copy_fn(ROWS0 + (s - 1) * ROWSR, ROWSR)

    _seed_writeback(lambda off, nrows: pltpu.sync_copy(
        t_hbm.at[pl.ds(off, nrows)], acc.at[pl.ds(off, nrows)]))
    pltpu.sync_copy(dst_hbm.at[wid], didx)
    plsc.subcore_barrier()

    def _gather(i, slot):
        pltpu.sync_copy(src_hbm.at[pl.ds(wid * EPT + i * CHUNK, CHUNK)],
                        sidx[slot])
        pltpu.async_copy(t_hbm.at[sidx[slot]], rows[slot], gsem[slot])

    def _gwait(slot):
        pltpu.make_async_copy(t_hbm.at[sidx[slot]], rows[slot],
                              gsem[slot]).wait()

    def _scat(i, slot):
        pltpu.async_copy(rows[slot], acc.at[didx.at[i]], ssem[slot], add=True)

    def _swait(slot):
        pltpu.make_async_copy(rows[slot], acc.at[didx.at[0]],
                              ssem[slot]).wait()

    _gather(0, 0)
    _gwait(0)
    _gather(1, 1)
    _scat(0, 0)

    def pair(j, carry):
        i0 = 2 * j
        _gwait(1)
        _swait(0)
        _gather(i0, 0)
        _scat(i0 - 1, 1)
        _gwait(0)
        _swait(1)
        _gather(i0 + 1, 1)
        _scat(i0, 0)
        return carry

    lax.fori_loop(1, (NCHUNK - 1) // 2, pair, 0)
    _gwait(1)
    _swait(0)
    _gather(NCHUNK - 1, 0)
    _scat(NCHUNK - 2, 1)
    _gwait(0)
    _swait(1)
    _scat(NCHUNK - 1, 0)
    _swait(0)
    plsc.subcore_barrier()
    _seed_writeback(lambda off, nrows: pltpu.sync_copy(
        acc.at[pl.ds(off, nrows)], out_hbm.at[pl.ds(c * N + off, nrows)]))


@functools.lru_cache(maxsize=None)
def _sc_calls():
    mesh = plsc.VectorSubcoreMesh(
        core_axis_name="c", subcore_axis_name="s",
        num_cores=NC, num_subcores=NS)
    deg_call = pl.kernel(
        _deg_body,
        out_type=jax.ShapeDtypeStruct((NW, N), jnp.float32),
        mesh=mesh,
        scratch_types=[
            pltpu.VMEM((EPT,), jnp.int32),
            pltpu.VMEM((N,), jnp.float32),
        ],
        compiler_params=pltpu.CompilerParams(needs_layout_passes=False),
    )
    edge_call = pl.kernel(
        _edge_body,
        out_type=jax.ShapeDtypeStruct((2 * N, D), jnp.float32),
        mesh=mesh,
        scratch_types=[
            [pltpu.VMEM((CHUNK,), jnp.int32) for _ in range(2)],
            pltpu.VMEM((NCHUNK, CHUNK), jnp.int32),
            [pltpu.VMEM((CHUNK, D), jnp.float32) for _ in range(2)],
            pltpu.VMEM_SHARED((N, D), jnp.float32),
            [pltpu.SemaphoreType.DMA for _ in range(2)],
            [pltpu.SemaphoreType.DMA for _ in range(2)],
        ],
    )
    return deg_call, edge_call


def _dinv_body(parts_ref, out_ref):
    deg = jnp.sum(parts_ref[...], axis=0, keepdims=True) + 1.0
    out_ref[...] = lax.rsqrt(deg)


def _proj_body(x_ref, w_ref, b_ref, dinv_ref, h_ref, t_ref):
    h = jnp.dot(x_ref[...], w_ref[...],
                preferred_element_type=jnp.float32,
                precision=lax.Precision.HIGHEST)
    h = jnp.maximum(h + b_ref[...], 0.0)
    h_ref[...] = h
    t_ref[...] = h * dinv_ref[...]


def _layer1_body(p0_ref, p1_ref, t_ref, x0_ref, dinv_ref, w_ref, o_ref):
    dinv = dinv_ref[...]
    s = p0_ref[...] + p1_ref[...] - t_ref[...]
    hm = (1.0 - ALPHA) * (dinv * s) + ALPHA * x0_ref[...]
    cur = jnp.maximum(
        jnp.dot(hm, w_ref[...], preferred_element_type=jnp.float32,
                precision=lax.Precision.HIGHEST), 0.0)
    o_ref[...] = dinv * cur


def _layer2_body(p0_ref, p1_ref, t_ref, x0_ref, dinv_ref, w_ref, wo_ref,
                 bo_ref, o_ref):
    dinv = dinv_ref[...]
    s = p0_ref[...] + p1_ref[...] - t_ref[...]
    hm = (1.0 - ALPHA) * (dinv * s) + ALPHA * x0_ref[...]
    cur = jnp.maximum(
        jnp.dot(hm, w_ref[...], preferred_element_type=jnp.float32,
                precision=lax.Precision.HIGHEST), 0.0)
    o_ref[...] = jnp.dot(cur, wo_ref[...], preferred_element_type=jnp.float32,
                         precision=lax.Precision.HIGHEST) + bo_ref[...]


def _row_spec(i_map=lambda i: (i, 0)):
    return pl.BlockSpec((RB, D), i_map)


_full_w = pl.BlockSpec((D, D), lambda i: (0, 0))
_full_b = pl.BlockSpec((1, D), lambda i: (0, 0))
_dinv_spec = pl.BlockSpec((RB, 1), lambda i: (i, 0))

_proj_call = pl.pallas_call(
    _proj_body,
    grid=(GRID,),
    in_specs=[_row_spec(), _full_w, _full_b, _dinv_spec],
    out_specs=[_row_spec(), _row_spec()],
    out_shape=[jax.ShapeDtypeStruct((N, D), jnp.float32),
               jax.ShapeDtypeStruct((N, D), jnp.float32)],
)

_layer1_call = pl.pallas_call(
    _layer1_body,
    grid=(GRID,),
    in_specs=[_row_spec(), _row_spec(lambda i: (GRID + i, 0)), _row_spec(),
              _row_spec(), _dinv_spec, _full_w],
    out_specs=_row_spec(),
    out_shape=jax.ShapeDtypeStruct((N, D), jnp.float32),
)

_layer2_call = pl.pallas_call(
    _layer2_body,
    grid=(GRID,),
    in_specs=[_row_spec(), _row_spec(lambda i: (GRID + i, 0)), _row_spec(),
              _row_spec(), _dinv_spec, _full_w, _full_w, _full_b],
    out_specs=_row_spec(),
    out_shape=jax.ShapeDtypeStruct((N, D), jnp.float32),
)

_dinv_call = pl.pallas_call(
    _dinv_body,
    out_shape=jax.ShapeDtypeStruct((1, N), jnp.float32),
)


def kernel(x, edge_index, W_in, b_in, W_convs, W_out, b_out):
    src = edge_index[0].astype(jnp.int32)
    dst = edge_index[1].astype(jnp.int32)
    _deg_call, _edge_call = _sc_calls()

    deg_parts = _deg_call(dst)
    dinv2d = _dinv_call(deg_parts)
    dinv_col = dinv2d.reshape(N, 1)

    h, t0 = _proj_call(x, W_in, b_in.reshape(1, D), dinv_col)

    dst3d = dst.reshape(NW, NCHUNK, CHUNK)
    p1 = _edge_call(t0, src, dst3d)
    t1 = _layer1_call(p1, p1, t0, h, dinv_col, W_convs[0])

    p2 = _edge_call(t1, src, dst3d)
    y = _layer2_call(p2, p2, t1, h, dinv_col, W_convs[1], W_out,
                     b_out.reshape(1, D))
    return y


# speedup vs baseline: 18.9771x; 1.0031x over previous
import functools

import jax
import jax.numpy as jnp
from jax import lax
from jax.experimental import pallas as pl
from jax.experimental.pallas import tpu as pltpu
from jax.experimental.pallas import tpu_sc as plsc

N = 10000
E = 320000
D = 128
ALPHA = 0.1

NC = 2
NS = 16
NW = NC * NS
EPT = E // NW
CHUNK = 80
NCHUNK = EPT // CHUNK
RPT = N // NS
ROWS0 = 640
ROWSR = (N - ROWS0) // (NS - 1)
RB = 1000
GRID = N // RB

def _deg_body(dst_hbm, out_hbm, dstbuf, degbuf):
    c = lax.axis_index("c")
    s = lax.axis_index("s")
    wid = c * NS + s
    pltpu.sync_copy(dst_hbm.at[pl.ds(wid * EPT, EPT)], dstbuf)

    def zero(i, carry):
        degbuf[pl.ds(i * 16, 16)] = jnp.zeros((16,), jnp.float32)
        return carry

    lax.fori_loop(0, N // 16, zero, 0)
    ones = jnp.ones((16,), jnp.float32)

    def count(i, carry):
        dv = dstbuf[pl.ds(i * 16, 16)]
        plsc.addupdate_scatter(degbuf, [dv], ones)
        return carry

    lax.fori_loop(0, EPT // 16, count, 0)
    pltpu.sync_copy(degbuf, out_hbm.at[wid])


def _edge_body(t_hbm, src_hbm, dst_hbm, out_hbm, sidx, didx, rows, acc, gsem,
               ssem):
    c = lax.axis_index("c")
    s = lax.axis_index("s")
    wid = c * NS + s

    def _seed_writeback(copy_fn):
        @pl.when(s == 0)
        def _():
            copy_fn(0, ROWS0)

        @pl.when(s > 0)
        def _():
            copy_fn(ROWS0 + (s - 1) * ROWSR, ROWSR)

    _seed_writeback(lambda off, nrows: pltpu.sync_copy(
        t_hbm.at[pl.ds(off, nrows)], acc.at[pl.ds(off, nrows)]))
    pltpu.sync_copy(dst_hbm.at[wid], didx)
    plsc.subcore_barrier()

    def _gather(i, slot):
        pltpu.sync_copy(src_hbm.at[pl.ds(wid * EPT + i * CHUNK, CHUNK)],
                        sidx[slot])
        pltpu.async_copy(t_hbm.at[sidx[slot]], rows[slot], gsem[slot])

    def _gwait(slot):
        pltpu.make_async_copy(t_hbm.at[sidx[slot]], rows[slot],
                              gsem[slot]).wait()

    def _scat(i, slot):
        pltpu.async_copy(rows[slot], acc.at[didx.at[i]], ssem[slot], add=True)

    def _swait(slot):
        pltpu.make_async_copy(rows[slot], acc.at[didx.at[0]],
                              ssem[slot]).wait()

    _gather(0, 0)
    _gwait(0)
    _gather(1, 1)
    _scat(0, 0)

    def pair(j, carry):
        i0 = 2 * j
        _gwait(1)
        _swait(0)
        _gather(i0, 0)
        _scat(i0 - 1, 1)
        _gwait(0)
        _swait(1)
        _gather(i0 + 1, 1)
        _scat(i0, 0)
        return carry

    lax.fori_loop(1, (NCHUNK - 1) // 2, pair, 0)
    _gwait(1)
    _swait(0)
    _gather(NCHUNK - 1, 0)
    _scat(NCHUNK - 2, 1)
    _gwait(0)
    _swait(1)
    _scat(NCHUNK - 1, 0)
    _swait(0)
    plsc.subcore_barrier()
    _seed_writeback(lambda off, nrows: pltpu.sync_copy(
        acc.at[pl.ds(off, nrows)], out_hbm.at[pl.ds(c * N + off, nrows)]))


@functools.lru_cache(maxsize=None)
def _sc_calls():
    mesh = plsc.VectorSubcoreMesh(
        core_axis_name="c", subcore_axis_name="s",
        num_cores=NC, num_subcores=NS)
    deg_call = pl.kernel(
        _deg_body,
        out_type=jax.ShapeDtypeStruct((NW, N), jnp.float32),
        mesh=mesh,
        scratch_types=[
            pltpu.VMEM((EPT,), jnp.int32),
            pltpu.VMEM((N,), jnp.float32),
        ],
        compiler_params=pltpu.CompilerParams(needs_layout_passes=False),
    )
    edge_call = pl.kernel(
        _edge_body,
        out_type=jax.ShapeDtypeStruct((2 * N, D), jnp.float32),
        mesh=mesh,
        scratch_types=[
            [pltpu.VMEM((CHUNK,), jnp.int32) for _ in range(2)],
            pltpu.VMEM((NCHUNK, CHUNK), jnp.int32),
            [pltpu.VMEM((CHUNK, D), jnp.float32) for _ in range(2)],
            pltpu.VMEM_SHARED((N, D), jnp.float32),
            [pltpu.SemaphoreType.DMA for _ in range(2)],
            [pltpu.SemaphoreType.DMA for _ in range(2)],
        ],
    )
    return deg_call, edge_call


def _dinv_body(parts_ref, out_ref):
    deg = jnp.sum(parts_ref[...], axis=0) + 1.0
    out_ref[...] = lax.rsqrt(deg)[:, None]


def _proj_body(x_ref, w_ref, b_ref, dinv_ref, h_ref, t_ref):
    h = jnp.dot(x_ref[...], w_ref[...],
                preferred_element_type=jnp.float32,
                precision=lax.Precision.HIGHEST)
    h = jnp.maximum(h + b_ref[...], 0.0)
    h_ref[...] = h
    t_ref[...] = h * dinv_ref[...]


def _layer1_body(p0_ref, p1_ref, t_ref, x0_ref, dinv_ref, w_ref, o_ref):
    dinv = dinv_ref[...]
    s = p0_ref[...] + p1_ref[...] - t_ref[...]
    hm = (1.0 - ALPHA) * (dinv * s) + ALPHA * x0_ref[...]
    cur = jnp.maximum(
        jnp.dot(hm, w_ref[...], preferred_element_type=jnp.float32,
                precision=lax.Precision.HIGHEST), 0.0)
    o_ref[...] = dinv * cur


def _layer2_body(p0_ref, p1_ref, t_ref, x0_ref, dinv_ref, w_ref, wo_ref,
                 bo_ref, o_ref):
    dinv = dinv_ref[...]
    s = p0_ref[...] + p1_ref[...] - t_ref[...]
    hm = (1.0 - ALPHA) * (dinv * s) + ALPHA * x0_ref[...]
    cur = jnp.maximum(
        jnp.dot(hm, w_ref[...], preferred_element_type=jnp.float32,
                precision=lax.Precision.HIGHEST), 0.0)
    o_ref[...] = jnp.dot(cur, wo_ref[...], preferred_element_type=jnp.float32,
                         precision=lax.Precision.HIGHEST) + bo_ref[...]


def _row_spec(i_map=lambda i: (i, 0)):
    return pl.BlockSpec((RB, D), i_map)


_full_w = pl.BlockSpec((D, D), lambda i: (0, 0))
_full_b = pl.BlockSpec((1, D), lambda i: (0, 0))
_dinv_spec = pl.BlockSpec((RB, 1), lambda i: (i, 0))

_dinv_call = pl.pallas_call(
    _dinv_body,
    out_shape=jax.ShapeDtypeStruct((N, 1), jnp.float32),
)

_proj_call = pl.pallas_call(
    _proj_body,
    grid=(GRID,),
    in_specs=[_row_spec(), _full_w, _full_b, _dinv_spec],
    out_specs=[_row_spec(), _row_spec()],
    out_shape=[jax.ShapeDtypeStruct((N, D), jnp.float32),
               jax.ShapeDtypeStruct((N, D), jnp.float32)],
)

_layer1_call = pl.pallas_call(
    _layer1_body,
    grid=(GRID,),
    in_specs=[_row_spec(), _row_spec(lambda i: (GRID + i, 0)), _row_spec(),
              _row_spec(), _dinv_spec, _full_w],
    out_specs=_row_spec(),
    out_shape=jax.ShapeDtypeStruct((N, D), jnp.float32),
)

_layer2_call = pl.pallas_call(
    _layer2_body,
    grid=(GRID,),
    in_specs=[_row_spec(), _row_spec(lambda i: (GRID + i, 0)), _row_spec(),
              _row_spec(), _dinv_spec, _full_w, _full_w, _full_b],
    out_specs=_row_spec(),
    out_shape=jax.ShapeDtypeStruct((N, D), jnp.float32),
)

def kernel(x, edge_index, W_in, b_in, W_convs, W_out, b_out):
    src = edge_index[0].astype(jnp.int32)
    dst = edge_index[1].astype(jnp.int32)
    _deg_call, _edge_call = _sc_calls()

    deg_parts = _deg_call(dst)
    dinv_col = _dinv_call(deg_parts)
    h, t0 = _proj_call(x, W_in, b_in.reshape(1, D), dinv_col)

    dst3d = dst.reshape(NW, NCHUNK, CHUNK)
    p1 = _edge_call(t0, src, dst3d)
    t1 = _layer1_call(p1, p1, t0, h, dinv_col, W_convs[0])

    p2 = _edge_call(t1, src, dst3d)
    y = _layer2_call(p2, p2, t1, h, dinv_col, W_convs[1], W_out,
                     b_out.reshape(1, D))
    return y


# speedup vs baseline: 23.4457x; 1.2355x over previous
import functools

import jax
import jax.numpy as jnp
from jax import lax
from jax.experimental import pallas as pl
from jax.experimental.pallas import tpu as pltpu
from jax.experimental.pallas import tpu_sc as plsc

N = 10000
E = 320000
D = 128
ALPHA = 0.1

NC = 2
NS = 16
NW = NC * NS
EPT = E // NW
CHUNK = 80
NCHUNK = EPT // CHUNK
RPT = N // NS
ROWS0 = 640
ROWSR = (N - ROWS0) // (NS - 1)
RB = 1000
GRID = N // RB

def _deg_body(dst_hbm, out_hbm, dstbuf, degbuf):
    c = lax.axis_index("c")
    s = lax.axis_index("s")
    wid = c * NS + s
    pltpu.sync_copy(dst_hbm.at[pl.ds(wid * EPT, EPT)], dstbuf)

    def zero(i, carry):
        degbuf[pl.ds(i * 16, 16)] = jnp.zeros((16,), jnp.float32)
        return carry

    lax.fori_loop(0, N // 16, zero, 0)
    ones = jnp.ones((16,), jnp.float32)

    def count(i, carry):
        dv = dstbuf[pl.ds(i * 16, 16)]
        plsc.addupdate_scatter(degbuf, [dv], ones)
        return carry

    lax.fori_loop(0, EPT // 16, count, 0)
    pltpu.sync_copy(degbuf, out_hbm.at[wid])


def _edge_body(t_hbm, src_hbm, dst_hbm, out_hbm, sidx, didx, rows, acc, gsem,
               ssem, isem):
    c = lax.axis_index("c")
    s = lax.axis_index("s")
    wid = c * NS + s

    def _seed_writeback(copy_fn):
        @pl.when(s == 0)
        def _():
            copy_fn(0, ROWS0)

        @pl.when(s > 0)
        def _():
            copy_fn(ROWS0 + (s - 1) * ROWSR, ROWSR)

    _seed_writeback(lambda off, nrows: pltpu.sync_copy(
        t_hbm.at[pl.ds(off, nrows)], acc.at[pl.ds(off, nrows)]))
    pltpu.sync_copy(dst_hbm.at[wid], didx)
    plsc.subcore_barrier()

    def _stage(i, slot):
        pltpu.async_copy(src_hbm.at[pl.ds(wid * EPT + i * CHUNK, CHUNK)],
                         sidx[slot], isem[slot])

    def _gather(i, slot):
        pltpu.make_async_copy(src_hbm.at[pl.ds(wid * EPT, CHUNK)],
                              sidx[slot], isem[slot]).wait()
        pltpu.async_copy(t_hbm.at[sidx[slot]], rows[slot], gsem[slot])

    def _gwait(slot):
        pltpu.make_async_copy(t_hbm.at[sidx[slot]], rows[slot],
                              gsem[slot]).wait()

    def _scat(i, slot):
        pltpu.async_copy(rows[slot], acc.at[didx.at[i]], ssem[slot], add=True)

    def _swait(slot):
        pltpu.make_async_copy(rows[slot], acc.at[didx.at[0]],
                              ssem[slot]).wait()

    _stage(0, 0)
    _stage(1, 1)
    _gather(0, 0)
    _gwait(0)
    _stage(2, 0)
    _gather(1, 1)
    _scat(0, 0)

    def pair(j, carry):
        i0 = 2 * j
        _gwait(1)
        _stage(i0 + 1, 1)
        _swait(0)
        _gather(i0, 0)
        _scat(i0 - 1, 1)
        _gwait(0)
        _stage(i0 + 2, 0)
        _swait(1)
        _gather(i0 + 1, 1)
        _scat(i0, 0)
        return carry

    lax.fori_loop(1, (NCHUNK - 1) // 2, pair, 0)
    _gwait(1)
    _swait(0)
    _gather(NCHUNK - 1, 0)
    _scat(NCHUNK - 2, 1)
    _gwait(0)
    _swait(1)
    _scat(NCHUNK - 1, 0)
    _swait(0)
    plsc.subcore_barrier()
    _seed_writeback(lambda off, nrows: pltpu.sync_copy(
        acc.at[pl.ds(off, nrows)], out_hbm.at[pl.ds(c * N + off, nrows)]))


@functools.lru_cache(maxsize=None)
def _sc_calls():
    mesh = plsc.VectorSubcoreMesh(
        core_axis_name="c", subcore_axis_name="s",
        num_cores=NC, num_subcores=NS)
    deg_call = pl.kernel(
        _deg_body,
        out_type=jax.ShapeDtypeStruct((NW, N), jnp.float32),
        mesh=mesh,
        scratch_types=[
            pltpu.VMEM((EPT,), jnp.int32),
            pltpu.VMEM((N,), jnp.float32),
        ],
        compiler_params=pltpu.CompilerParams(needs_layout_passes=False),
    )
    edge_call = pl.kernel(
        _edge_body,
        out_type=jax.ShapeDtypeStruct((2 * N, D), jnp.float32),
        mesh=mesh,
        scratch_types=[
            [pltpu.VMEM((CHUNK,), jnp.int32) for _ in range(2)],
            pltpu.VMEM((NCHUNK, CHUNK), jnp.int32),
            [pltpu.VMEM((CHUNK, D), jnp.float32) for _ in range(2)],
            pltpu.VMEM_SHARED((N, D), jnp.float32),
            [pltpu.SemaphoreType.DMA for _ in range(2)],
            [pltpu.SemaphoreType.DMA for _ in range(2)],
            [pltpu.SemaphoreType.DMA for _ in range(2)],
        ],
    )
    return deg_call, edge_call


def _dinv_body(parts_ref, out_ref):
    deg = jnp.sum(parts_ref[...], axis=0) + 1.0
    out_ref[...] = lax.rsqrt(deg)[:, None]


def _proj_body(x_ref, w_ref, b_ref, dinv_ref, h_ref, t_ref):
    h = jnp.dot(x_ref[...], w_ref[...],
                preferred_element_type=jnp.float32,
                precision=lax.Precision.HIGHEST)
    h = jnp.maximum(h + b_ref[...], 0.0)
    h_ref[...] = h
    t_ref[...] = h * dinv_ref[...]


def _layer1_body(p0_ref, p1_ref, t_ref, x0_ref, dinv_ref, w_ref, o_ref):
    dinv = dinv_ref[...]
    s = p0_ref[...] + p1_ref[...] - t_ref[...]
    hm = (1.0 - ALPHA) * (dinv * s) + ALPHA * x0_ref[...]
    cur = jnp.maximum(
        jnp.dot(hm, w_ref[...], preferred_element_type=jnp.float32,
                precision=lax.Precision.HIGHEST), 0.0)
    o_ref[...] = dinv * cur


def _layer2_body(p0_ref, p1_ref, t_ref, x0_ref, dinv_ref, w_ref, wo_ref,
                 bo_ref, o_ref):
    dinv = dinv_ref[...]
    s = p0_ref[...] + p1_ref[...] - t_ref[...]
    hm = (1.0 - ALPHA) * (dinv * s) + ALPHA * x0_ref[...]
    cur = jnp.maximum(
        jnp.dot(hm, w_ref[...], preferred_element_type=jnp.float32,
                precision=lax.Precision.HIGHEST), 0.0)
    o_ref[...] = jnp.dot(cur, wo_ref[...], preferred_element_type=jnp.float32,
                         precision=lax.Precision.HIGHEST) + bo_ref[...]


def _row_spec(i_map=lambda i: (i, 0)):
    return pl.BlockSpec((RB, D), i_map)


_full_w = pl.BlockSpec((D, D), lambda i: (0, 0))
_full_b = pl.BlockSpec((1, D), lambda i: (0, 0))
_dinv_spec = pl.BlockSpec((RB, 1), lambda i: (i, 0))

_dinv_call = pl.pallas_call(
    _dinv_body,
    out_shape=jax.ShapeDtypeStruct((N, 1), jnp.float32),
)

_proj_call = pl.pallas_call(
    _proj_body,
    grid=(GRID,),
    in_specs=[_row_spec(), _full_w, _full_b, _dinv_spec],
    out_specs=[_row_spec(), _row_spec()],
    out_shape=[jax.ShapeDtypeStruct((N, D), jnp.float32),
               jax.ShapeDtypeStruct((N, D), jnp.float32)],
)

_layer1_call = pl.pallas_call(
    _layer1_body,
    grid=(GRID,),
    in_specs=[_row_spec(), _row_spec(lambda i: (GRID + i, 0)), _row_spec(),
              _row_spec(), _dinv_spec, _full_w],
    out_specs=_row_spec(),
    out_shape=jax.ShapeDtypeStruct((N, D), jnp.float32),
)

_layer2_call = pl.pallas_call(
    _layer2_body,
    grid=(GRID,),
    in_specs=[_row_spec(), _row_spec(lambda i: (GRID + i, 0)), _row_spec(),
              _row_spec(), _dinv_spec, _full_w, _full_w, _full_b],
    out_specs=_row_spec(),
    out_shape=jax.ShapeDtypeStruct((N, D), jnp.float32),
)

def kernel(x, edge_index, W_in, b_in, W_convs, W_out, b_out):
    src = edge_index[0].astype(jnp.int32)
    dst = edge_index[1].astype(jnp.int32)
    _deg_call, _edge_call = _sc_calls()

    deg_parts = _deg_call(dst)
    dinv_col = _dinv_call(deg_parts)
    h, t0 = _proj_call(x, W_in, b_in.reshape(1, D), dinv_col)

    dst3d = dst.reshape(NW, NCHUNK, CHUNK)
    p1 = _edge_call(t0, src, dst3d)
    t1 = _layer1_call(p1, p1, t0, h, dinv_col, W_convs[0])

    p2 = _edge_call(t1, src, dst3d)
    y = _layer2_call(p2, p2, t1, h, dinv_col, W_convs[1], W_out,
                     b_out.reshape(1, D))
    return y


# speedup vs baseline: 23.5437x; 1.0042x over previous
import functools

import jax
import jax.numpy as jnp
from jax import lax
from jax.experimental import pallas as pl
from jax.experimental.pallas import tpu as pltpu
from jax.experimental.pallas import tpu_sc as plsc

N = 10000
E = 320000
D = 128
ALPHA = 0.1

NC = 2
NS = 16
NW = NC * NS
EPT = E // NW
CHUNK = 80
NCHUNK = EPT // CHUNK
RPT = N // NS
ROWS0 = 640
ROWSR = (N - ROWS0) // (NS - 1)
RB = 1000
GRID = N // RB

def _deg_body(dst_hbm, out_hbm, dstbuf, degbuf):
    c = lax.axis_index("c")
    s = lax.axis_index("s")
    wid = c * NS + s
    pltpu.sync_copy(dst_hbm.at[pl.ds(wid * EPT, EPT)], dstbuf)

    def zero(i, carry):
        degbuf[pl.ds(i * 16, 16)] = jnp.zeros((16,), jnp.float32)
        return carry

    lax.fori_loop(0, N // 16, zero, 0)
    ones = jnp.ones((16,), jnp.float32)

    def count(i, carry):
        dv = dstbuf[pl.ds(i * 16, 16)]
        plsc.addupdate_scatter(degbuf, [dv], ones)
        return carry

    lax.fori_loop(0, EPT // 16, count, 0)
    pltpu.sync_copy(degbuf, out_hbm.at[wid])


def _edge_body(t_hbm, src_hbm, dst_hbm, out_hbm, sidx, didx, rows, acc, gsem,
               ssem, isem):
    c = lax.axis_index("c")
    s = lax.axis_index("s")
    wid = c * NS + s

    def _seed_writeback(copy_fn):
        @pl.when(s == 0)
        def _():
            copy_fn(0, ROWS0)

        @pl.when(s > 0)
        def _():
            copy_fn(ROWS0 + (s - 1) * ROWSR, ROWSR)

    _seed_writeback(lambda off, nrows: pltpu.sync_copy(
        t_hbm.at[pl.ds(off, nrows)], acc.at[pl.ds(off, nrows)]))
    pltpu.sync_copy(dst_hbm.at[wid], didx)
    plsc.subcore_barrier()

    def _stage(i, slot):
        pltpu.async_copy(src_hbm.at[pl.ds(wid * EPT + i * CHUNK, CHUNK)],
                         sidx[slot], isem[slot])

    def _gather(i, slot):
        pltpu.make_async_copy(src_hbm.at[pl.ds(wid * EPT, CHUNK)],
                              sidx[slot], isem[slot]).wait()
        pltpu.async_copy(t_hbm.at[sidx[slot]], rows[slot], gsem[slot])

    def _gwait(slot):
        pltpu.make_async_copy(t_hbm.at[sidx[slot]], rows[slot],
                              gsem[slot]).wait()

    def _scat(i, slot):
        pltpu.async_copy(rows[slot], acc.at[didx.at[i]], ssem[slot], add=True)

    def _swait(slot):
        pltpu.make_async_copy(rows[slot], acc.at[didx.at[0]],
                              ssem[slot]).wait()

    _stage(0, 0)
    _stage(1, 1)
    _gather(0, 0)
    _gwait(0)
    _stage(2, 0)
    _gather(1, 1)
    _scat(0, 0)

    def pair(j, carry):
        i0 = 2 * j
        _gwait(1)
        _stage(i0 + 1, 1)
        _swait(0)
        _gather(i0, 0)
        _scat(i0 - 1, 1)
        _gwait(0)
        _stage(i0 + 2, 0)
        _swait(1)
        _gather(i0 + 1, 1)
        _scat(i0, 0)
        return carry

    lax.fori_loop(1, (NCHUNK - 1) // 2, pair, 0)
    _gwait(1)
    _swait(0)
    _gather(NCHUNK - 1, 0)
    _scat(NCHUNK - 2, 1)
    _gwait(0)
    _swait(1)
    _scat(NCHUNK - 1, 0)
    _swait(0)
    plsc.subcore_barrier()
    _seed_writeback(lambda off, nrows: pltpu.sync_copy(
        acc.at[pl.ds(off, nrows)], out_hbm.at[pl.ds(c * N + off, nrows)]))


@functools.lru_cache(maxsize=None)
def _sc_calls():
    mesh = plsc.VectorSubcoreMesh(
        core_axis_name="c", subcore_axis_name="s",
        num_cores=NC, num_subcores=NS)
    deg_call = pl.kernel(
        _deg_body,
        out_type=jax.ShapeDtypeStruct((NW, N), jnp.float32),
        mesh=mesh,
        scratch_types=[
            pltpu.VMEM((EPT,), jnp.int32),
            pltpu.VMEM((N,), jnp.float32),
        ],
        compiler_params=pltpu.CompilerParams(needs_layout_passes=False),
    )
    edge_call = pl.kernel(
        _edge_body,
        out_type=jax.ShapeDtypeStruct((2 * N, D), jnp.float32),
        mesh=mesh,
        scratch_types=[
            [pltpu.VMEM((CHUNK,), jnp.int32) for _ in range(2)],
            pltpu.VMEM((NCHUNK, CHUNK), jnp.int32),
            [pltpu.VMEM((CHUNK, D), jnp.float32) for _ in range(2)],
            pltpu.VMEM_SHARED((N, D), jnp.float32),
            [pltpu.SemaphoreType.DMA for _ in range(2)],
            [pltpu.SemaphoreType.DMA for _ in range(2)],
            [pltpu.SemaphoreType.DMA for _ in range(2)],
        ],
    )
    return deg_call, edge_call


def _proj_body(parts_ref, x_ref, w_ref, b_ref, h_ref, t_ref, dinv_ref):
    i = pl.program_id(0)

    @pl.when(i == 0)
    def _():
        deg = jnp.sum(parts_ref[...], axis=0) + 1.0
        dinv_ref[...] = lax.rsqrt(deg)[:, None]

    dinv = dinv_ref[pl.ds(pl.multiple_of(i * RB, 8), RB), :]
    h = jnp.dot(x_ref[...], w_ref[...],
                preferred_element_type=jnp.float32,
                precision=lax.Precision.HIGHEST)
    h = jnp.maximum(h + b_ref[...], 0.0)
    h_ref[...] = h
    t_ref[...] = h * dinv


def _layer1_body(p0_ref, p1_ref, t_ref, x0_ref, dinv_ref, w_ref, o_ref):
    dinv = dinv_ref[...]
    s = p0_ref[...] + p1_ref[...] - t_ref[...]
    hm = (1.0 - ALPHA) * (dinv * s) + ALPHA * x0_ref[...]
    cur = jnp.maximum(
        jnp.dot(hm, w_ref[...], preferred_element_type=jnp.float32,
                precision=lax.Precision.HIGHEST), 0.0)
    o_ref[...] = dinv * cur


def _layer2_body(p0_ref, p1_ref, t_ref, x0_ref, dinv_ref, w_ref, wo_ref,
                 bo_ref, o_ref):
    dinv = dinv_ref[...]
    s = p0_ref[...] + p1_ref[...] - t_ref[...]
    hm = (1.0 - ALPHA) * (dinv * s) + ALPHA * x0_ref[...]
    cur = jnp.maximum(
        jnp.dot(hm, w_ref[...], preferred_element_type=jnp.float32,
                precision=lax.Precision.HIGHEST), 0.0)
    o_ref[...] = jnp.dot(cur, wo_ref[...], preferred_element_type=jnp.float32,
                         precision=lax.Precision.HIGHEST) + bo_ref[...]


def _row_spec(i_map=lambda i: (i, 0)):
    return pl.BlockSpec((RB, D), i_map)


_full_w = pl.BlockSpec((D, D), lambda i: (0, 0))
_full_b = pl.BlockSpec((1, D), lambda i: (0, 0))
_dinv_spec = pl.BlockSpec((RB, 1), lambda i: (i, 0))

_proj_call = pl.pallas_call(
    _proj_body,
    grid=(GRID,),
    in_specs=[pl.BlockSpec((NW, N), lambda i: (0, 0)), _row_spec(),
              _full_w, _full_b],
    out_specs=[_row_spec(), _row_spec(), pl.BlockSpec((N, 1), lambda i: (0, 0))],
    out_shape=[jax.ShapeDtypeStruct((N, D), jnp.float32),
               jax.ShapeDtypeStruct((N, D), jnp.float32),
               jax.ShapeDtypeStruct((N, 1), jnp.float32)],
)

_layer1_call = pl.pallas_call(
    _layer1_body,
    grid=(GRID,),
    in_specs=[_row_spec(), _row_spec(lambda i: (GRID + i, 0)), _row_spec(),
              _row_spec(), _dinv_spec, _full_w],
    out_specs=_row_spec(),
    out_shape=jax.ShapeDtypeStruct((N, D), jnp.float32),
)

_layer2_call = pl.pallas_call(
    _layer2_body,
    grid=(GRID,),
    in_specs=[_row_spec(), _row_spec(lambda i: (GRID + i, 0)), _row_spec(),
              _row_spec(), _dinv_spec, _full_w, _full_w, _full_b],
    out_specs=_row_spec(),
    out_shape=jax.ShapeDtypeStruct((N, D), jnp.float32),
)

def kernel(x, edge_index, W_in, b_in, W_convs, W_out, b_out):
    src = edge_index[0].astype(jnp.int32)
    dst = edge_index[1].astype(jnp.int32)
    _deg_call, _edge_call = _sc_calls()

    deg_parts = _deg_call(dst)
    h, t0, dinv_col = _proj_call(deg_parts, x, W_in, b_in.reshape(1, D))

    dst3d = dst.reshape(NW, NCHUNK, CHUNK)
    p1 = _edge_call(t0, src, dst3d)
    t1 = _layer1_call(p1, p1, t0, h, dinv_col, W_convs[0])

    p2 = _edge_call(t1, src, dst3d)
    y = _layer2_call(p2, p2, t1, h, dinv_col, W_convs[1], W_out,
                     b_out.reshape(1, D))
    return y


# speedup vs baseline: 23.5872x; 1.0018x over previous
import functools

import jax
import jax.numpy as jnp
from jax import lax
from jax.experimental import pallas as pl
from jax.experimental.pallas import tpu as pltpu
from jax.experimental.pallas import tpu_sc as plsc

N = 10000
E = 320000
D = 128
ALPHA = 0.1

NC = 2
NS = 16
NW = NC * NS
EPT = E // NW
CHUNK = 80
NCHUNK = EPT // CHUNK
RPT = N // NS
ROWS0 = 640
ROWSR = (N - ROWS0) // (NS - 1)
RB = 1000
GRID = N // RB

def _deg_body(dst_hbm, out_hbm, dstbuf, degbuf):
    c = lax.axis_index("c")
    s = lax.axis_index("s")
    wid = c * NS + s
    pltpu.sync_copy(dst_hbm.at[pl.ds(wid * EPT, EPT)], dstbuf)

    def zero(i, carry):
        degbuf[pl.ds(i * 16, 16)] = jnp.zeros((16,), jnp.float32)
        return carry

    lax.fori_loop(0, N // 16, zero, 0)
    ones = jnp.ones((16,), jnp.float32)

    def count(i, carry):
        dv = dstbuf[pl.ds(i * 16, 16)]
        plsc.addupdate_scatter(degbuf, [dv], ones)
        return carry

    lax.fori_loop(0, EPT // 16, count, 0)
    pltpu.sync_copy(degbuf, out_hbm.at[wid])


def _edge_body(t_hbm, src_hbm, dst_hbm, out_hbm, sidx, didx, rows, acc, gsem,
               ssem, isem):
    c = lax.axis_index("c")
    s = lax.axis_index("s")
    wid = c * NS + s

    def _seed_writeback(copy_fn):
        @pl.when(s == 0)
        def _():
            copy_fn(0, ROWS0)

        @pl.when(s > 0)
        def _():
            copy_fn(ROWS0 + (s - 1) * ROWSR, ROWSR)

    _seed_writeback(lambda off, nrows: pltpu.sync_copy(
        t_hbm.at[pl.ds(off, nrows)], acc.at[pl.ds(off, nrows)]))
    pltpu.sync_copy(dst_hbm.at[wid], didx)
    plsc.subcore_barrier()

    def _stage(i, slot):
        pltpu.async_copy(src_hbm.at[pl.ds(wid * EPT + i * CHUNK, CHUNK)],
                         sidx[slot], isem[slot])

    def _gather(i, slot):
        pltpu.make_async_copy(src_hbm.at[pl.ds(wid * EPT, CHUNK)],
                              sidx[slot], isem[slot]).wait()
        pltpu.async_copy(t_hbm.at[sidx[slot]], rows[slot], gsem[slot])

    def _gwait(slot):
        pltpu.make_async_copy(t_hbm.at[sidx[slot]], rows[slot],
                              gsem[slot]).wait()

    def _scat(i, slot):
        pltpu.async_copy(rows[slot], acc.at[didx.at[i]], ssem[slot], add=True)

    def _swait(slot):
        pltpu.make_async_copy(rows[slot], acc.at[didx.at[0]],
                              ssem[slot]).wait()

    _stage(0, 0)
    _stage(1, 1)
    _gather(0, 0)
    _gwait(0)
    _stage(2, 0)
    _gather(1, 1)
    _scat(0, 0)

    def pair(j, carry):
        i0 = 2 * j
        _gwait(1)
        _stage(i0 + 1, 1)
        _swait(0)
        _gather(i0, 0)
        _scat(i0 - 1, 1)
        _gwait(0)
        _stage(i0 + 2, 0)
        _swait(1)
        _gather(i0 + 1, 1)
        _scat(i0, 0)
        return carry

    lax.fori_loop(1, (NCHUNK - 1) // 2, pair, 0)
    _gwait(1)
    _swait(0)
    _gather(NCHUNK - 1, 0)
    _scat(NCHUNK - 2, 1)
    _gwait(0)
    _swait(1)
    _scat(NCHUNK - 1, 0)
    _swait(0)
    plsc.subcore_barrier()
    _seed_writeback(lambda off, nrows: pltpu.sync_copy(
        acc.at[pl.ds(off, nrows)], out_hbm.at[pl.ds(c * N + off, nrows)]))


@functools.lru_cache(maxsize=None)
def _sc_calls():
    mesh = plsc.VectorSubcoreMesh(
        core_axis_name="c", subcore_axis_name="s",
        num_cores=NC, num_subcores=NS)
    deg_call = pl.kernel(
        _deg_body,
        out_type=jax.ShapeDtypeStruct((NW, N), jnp.float32),
        mesh=mesh,
        scratch_types=[
            pltpu.VMEM((EPT,), jnp.int32),
            pltpu.VMEM((N,), jnp.float32),
        ],
        compiler_params=pltpu.CompilerParams(needs_layout_passes=False, skip_device_barrier=True),
    )
    edge_call = pl.kernel(
        _edge_body,
        out_type=jax.ShapeDtypeStruct((2 * N, D), jnp.float32),
        mesh=mesh,
        scratch_types=[
            [pltpu.VMEM((CHUNK,), jnp.int32) for _ in range(2)],
            pltpu.VMEM((NCHUNK, CHUNK), jnp.int32),
            [pltpu.VMEM((CHUNK, D), jnp.float32) for _ in range(2)],
            pltpu.VMEM_SHARED((N, D), jnp.float32),
            [pltpu.SemaphoreType.DMA for _ in range(2)],
            [pltpu.SemaphoreType.DMA for _ in range(2)],
            [pltpu.SemaphoreType.DMA for _ in range(2)],
        ],
        compiler_params=pltpu.CompilerParams(skip_device_barrier=True),
    )
    return deg_call, edge_call


def _proj_body(parts_ref, x_ref, w_ref, b_ref, h_ref, t_ref, dinv_ref):
    i = pl.program_id(0)

    @pl.when(i == 0)
    def _():
        deg = jnp.sum(parts_ref[...], axis=0) + 1.0
        dinv_ref[...] = lax.rsqrt(deg)[:, None]

    dinv = dinv_ref[pl.ds(pl.multiple_of(i * RB, 8), RB), :]
    h = jnp.dot(x_ref[...], w_ref[...],
                preferred_element_type=jnp.float32,
                precision=lax.Precision.HIGHEST)
    h = jnp.maximum(h + b_ref[...], 0.0)
    h_ref[...] = h
    t_ref[...] = h * dinv


def _layer1_body(p0_ref, p1_ref, t_ref, x0_ref, dinv_ref, w_ref, o_ref):
    dinv = dinv_ref[...]
    s = p0_ref[...] + p1_ref[...] - t_ref[...]
    hm = (1.0 - ALPHA) * (dinv * s) + ALPHA * x0_ref[...]
    cur = jnp.maximum(
        jnp.dot(hm, w_ref[...], preferred_element_type=jnp.float32,
                precision=lax.Precision.HIGHEST), 0.0)
    o_ref[...] = dinv * cur


def _layer2_body(p0_ref, p1_ref, t_ref, x0_ref, dinv_ref, w_ref, wo_ref,
                 bo_ref, o_ref):
    dinv = dinv_ref[...]
    s = p0_ref[...] + p1_ref[...] - t_ref[...]
    hm = (1.0 - ALPHA) * (dinv * s) + ALPHA * x0_ref[...]
    cur = jnp.maximum(
        jnp.dot(hm, w_ref[...], preferred_element_type=jnp.float32,
                precision=lax.Precision.HIGHEST), 0.0)
    o_ref[...] = jnp.dot(cur, wo_ref[...], preferred_element_type=jnp.float32,
                         precision=lax.Precision.HIGHEST) + bo_ref[...]


def _row_spec(i_map=lambda i: (i, 0)):
    return pl.BlockSpec((RB, D), i_map)


_full_w = pl.BlockSpec((D, D), lambda i: (0, 0))
_full_b = pl.BlockSpec((1, D), lambda i: (0, 0))
_dinv_spec = pl.BlockSpec((RB, 1), lambda i: (i, 0))

_proj_call = pl.pallas_call(
    _proj_body,
    grid=(GRID,),
    in_specs=[pl.BlockSpec((NW, N), lambda i: (0, 0)), _row_spec(),
              _full_w, _full_b],
    out_specs=[_row_spec(), _row_spec(), pl.BlockSpec((N, 1), lambda i: (0, 0))],
    out_shape=[jax.ShapeDtypeStruct((N, D), jnp.float32),
               jax.ShapeDtypeStruct((N, D), jnp.float32),
               jax.ShapeDtypeStruct((N, 1), jnp.float32)],
)

_layer1_call = pl.pallas_call(
    _layer1_body,
    grid=(GRID,),
    in_specs=[_row_spec(), _row_spec(lambda i: (GRID + i, 0)), _row_spec(),
              _row_spec(), _dinv_spec, _full_w],
    out_specs=_row_spec(),
    out_shape=jax.ShapeDtypeStruct((N, D), jnp.float32),
)

_layer2_call = pl.pallas_call(
    _layer2_body,
    grid=(GRID,),
    in_specs=[_row_spec(), _row_spec(lambda i: (GRID + i, 0)), _row_spec(),
              _row_spec(), _dinv_spec, _full_w, _full_w, _full_b],
    out_specs=_row_spec(),
    out_shape=jax.ShapeDtypeStruct((N, D), jnp.float32),
)

def kernel(x, edge_index, W_in, b_in, W_convs, W_out, b_out):
    src = edge_index[0].astype(jnp.int32)
    dst = edge_index[1].astype(jnp.int32)
    _deg_call, _edge_call = _sc_calls()

    deg_parts = _deg_call(dst)
    h, t0, dinv_col = _proj_call(deg_parts, x, W_in, b_in.reshape(1, D))

    dst3d = dst.reshape(NW, NCHUNK, CHUNK)
    p1 = _edge_call(t0, src, dst3d)
    t1 = _layer1_call(p1, p1, t0, h, dinv_col, W_convs[0])

    p2 = _edge_call(t1, src, dst3d)
    y = _layer2_call(p2, p2, t1, h, dinv_col, W_convs[1], W_out,
                     b_out.reshape(1, D))
    return y


# speedup vs baseline: 23.7060x; 1.0050x over previous
import functools

import jax
import jax.numpy as jnp
from jax import lax
from jax.experimental import pallas as pl
from jax.experimental.pallas import tpu as pltpu
from jax.experimental.pallas import tpu_sc as plsc

N = 10000
E = 320000
D = 128
ALPHA = 0.1

NC = 2
NS = 16
NW = NC * NS
EPT = E // NW
CHUNK = 80
NCHUNK = EPT // CHUNK
RPT = N // NS
ROWS0 = 640
ROWSR = (N - ROWS0) // (NS - 1)
RB = 1000
GRID = N // RB

def _deg_body(dst_hbm, out_hbm, dstbuf, degbuf):
    c = lax.axis_index("c")
    s = lax.axis_index("s")
    wid = c * NS + s
    pltpu.sync_copy(dst_hbm.at[pl.ds(wid * EPT, EPT)], dstbuf)

    def zero(i, carry):
        degbuf[pl.ds(i * 16, 16)] = jnp.zeros((16,), jnp.float32)
        return carry

    lax.fori_loop(0, N // 16, zero, 0)
    ones = jnp.ones((16,), jnp.float32)

    def count(i, carry):
        dv = dstbuf[pl.ds(i * 16, 16)]
        plsc.addupdate_scatter(degbuf, [dv], ones)
        return carry

    lax.fori_loop(0, EPT // 16, count, 0)
    pltpu.sync_copy(degbuf, out_hbm.at[wid])


def _edge_body(t_hbm, u_hbm, src_hbm, dst_hbm, out_hbm, sidx, didx, rows,
               acc, gsem, ssem, isem):
    c = lax.axis_index("c")
    s = lax.axis_index("s")
    wid = c * NS + s

    def _seed_writeback(copy_fn):
        @pl.when(s == 0)
        def _():
            copy_fn(0, ROWS0)

        @pl.when(s > 0)
        def _():
            copy_fn(ROWS0 + (s - 1) * ROWSR, ROWSR)

    def _seed(off, nrows):
        @pl.when(c == 0)
        def _():
            pltpu.sync_copy(t_hbm.at[pl.ds(off, nrows)],
                            acc.at[pl.ds(off, nrows)])

        @pl.when(c == 1)
        def _():
            pltpu.sync_copy(u_hbm.at[pl.ds(off, nrows)],
                            acc.at[pl.ds(off, nrows)])

    _seed_writeback(_seed)
    pltpu.sync_copy(dst_hbm.at[wid], didx)
    plsc.subcore_barrier()

    def _stage(i, slot):
        pltpu.async_copy(src_hbm.at[pl.ds(wid * EPT + i * CHUNK, CHUNK)],
                         sidx[slot], isem[slot])

    def _gather(i, slot):
        pltpu.make_async_copy(src_hbm.at[pl.ds(wid * EPT, CHUNK)],
                              sidx[slot], isem[slot]).wait()
        pltpu.async_copy(t_hbm.at[sidx[slot]], rows[slot], gsem[slot])

    def _gwait(slot):
        pltpu.make_async_copy(t_hbm.at[sidx[slot]], rows[slot],
                              gsem[slot]).wait()

    def _scat(i, slot):
        pltpu.async_copy(rows[slot], acc.at[didx.at[i]], ssem[slot], add=True)

    def _swait(slot):
        pltpu.make_async_copy(rows[slot], acc.at[didx.at[0]],
                              ssem[slot]).wait()

    _stage(0, 0)
    _stage(1, 1)
    _gather(0, 0)
    _gwait(0)
    _stage(2, 0)
    _gather(1, 1)
    _scat(0, 0)

    def pair(j, carry):
        i0 = 2 * j
        _gwait(1)
        _stage(i0 + 1, 1)
        _swait(0)
        _gather(i0, 0)
        _scat(i0 - 1, 1)
        _gwait(0)
        _stage(i0 + 2, 0)
        _swait(1)
        _gather(i0 + 1, 1)
        _scat(i0, 0)
        return carry

    lax.fori_loop(1, (NCHUNK - 1) // 2, pair, 0)
    _gwait(1)
    _swait(0)
    _gather(NCHUNK - 1, 0)
    _scat(NCHUNK - 2, 1)
    _gwait(0)
    _swait(1)
    _scat(NCHUNK - 1, 0)
    _swait(0)
    plsc.subcore_barrier()
    _seed_writeback(lambda off, nrows: pltpu.sync_copy(
        acc.at[pl.ds(off, nrows)], out_hbm.at[pl.ds(c * N + off, nrows)]))


@functools.lru_cache(maxsize=None)
def _sc_calls():
    mesh = plsc.VectorSubcoreMesh(
        core_axis_name="c", subcore_axis_name="s",
        num_cores=NC, num_subcores=NS)
    deg_call = pl.kernel(
        _deg_body,
        out_type=jax.ShapeDtypeStruct((NW, N), jnp.float32),
        mesh=mesh,
        scratch_types=[
            pltpu.VMEM((EPT,), jnp.int32),
            pltpu.VMEM((N,), jnp.float32),
        ],
        compiler_params=pltpu.CompilerParams(needs_layout_passes=False, skip_device_barrier=True),
    )
    edge_call = pl.kernel(
        _edge_body,
        out_type=jax.ShapeDtypeStruct((2 * N, D), jnp.float32),
        mesh=mesh,
        scratch_types=[
            [pltpu.VMEM((CHUNK,), jnp.int32) for _ in range(2)],
            pltpu.VMEM((NCHUNK, CHUNK), jnp.int32),
            [pltpu.VMEM((CHUNK, D), jnp.float32) for _ in range(2)],
            pltpu.VMEM_SHARED((N, D), jnp.float32),
            [pltpu.SemaphoreType.DMA for _ in range(2)],
            [pltpu.SemaphoreType.DMA for _ in range(2)],
            [pltpu.SemaphoreType.DMA for _ in range(2)],
        ],
        compiler_params=pltpu.CompilerParams(skip_device_barrier=True),
    )
    return deg_call, edge_call


def _proj_body(parts_ref, x_ref, w_ref, b_ref, t_ref, u_ref, dinv_ref):
    i = pl.program_id(0)

    @pl.when(i == 0)
    def _():
        deg = jnp.sum(parts_ref[...], axis=0) + 1.0
        dinv_ref[...] = lax.rsqrt(deg)[:, None]

    dinv = dinv_ref[pl.ds(pl.multiple_of(i * RB, 8), RB), :]
    h = jnp.dot(x_ref[...], w_ref[...],
                preferred_element_type=jnp.float32,
                precision=lax.Precision.HIGHEST)
    h = jnp.maximum(h + b_ref[...], 0.0)
    t_ref[...] = h * dinv
    u_ref[...] = h / ((( 1.0 - ALPHA) / ALPHA) * dinv)


def _layer1_body(p0_ref, p1_ref, dinv_ref, w_ref, o_ref):
    dinv = dinv_ref[...]
    hm = (1.0 - ALPHA) * (dinv * (p0_ref[...] + p1_ref[...]))
    cur = jnp.maximum(
        jnp.dot(hm, w_ref[...], preferred_element_type=jnp.float32,
                precision=lax.Precision.HIGHEST), 0.0)
    o_ref[...] = dinv * cur


def _layer2_body(p0_ref, p1_ref, dinv_ref, w_ref, wo_ref, bo_ref, o_ref):
    dinv = dinv_ref[...]
    hm = (1.0 - ALPHA) * (dinv * (p0_ref[...] + p1_ref[...]))
    cur = jnp.maximum(
        jnp.dot(hm, w_ref[...], preferred_element_type=jnp.float32,
                precision=lax.Precision.HIGHEST), 0.0)
    o_ref[...] = jnp.dot(cur, wo_ref[...], preferred_element_type=jnp.float32,
                         precision=lax.Precision.HIGHEST) + bo_ref[...]


def _row_spec(i_map=lambda i: (i, 0)):
    return pl.BlockSpec((RB, D), i_map)


_full_w = pl.BlockSpec((D, D), lambda i: (0, 0))
_full_b = pl.BlockSpec((1, D), lambda i: (0, 0))
_dinv_spec = pl.BlockSpec((RB, 1), lambda i: (i, 0))

_proj_call = pl.pallas_call(
    _proj_body,
    grid=(GRID,),
    in_specs=[pl.BlockSpec((NW, N), lambda i: (0, 0)), _row_spec(),
              _full_w, _full_b],
    out_specs=[_row_spec(), _row_spec(), pl.BlockSpec((N, 1), lambda i: (0, 0))],
    out_shape=[jax.ShapeDtypeStruct((N, D), jnp.float32),
               jax.ShapeDtypeStruct((N, D), jnp.float32),
               jax.ShapeDtypeStruct((N, 1), jnp.float32)],
)

_layer1_call = pl.pallas_call(
    _layer1_body,
    grid=(GRID,),
    in_specs=[_row_spec(), _row_spec(lambda i: (GRID + i, 0)), _dinv_spec,
              _full_w],
    out_specs=_row_spec(),
    out_shape=jax.ShapeDtypeStruct((N, D), jnp.float32),
)

_layer2_call = pl.pallas_call(
    _layer2_body,
    grid=(GRID,),
    in_specs=[_row_spec(), _row_spec(lambda i: (GRID + i, 0)), _dinv_spec,
              _full_w, _full_w, _full_b],
    out_specs=_row_spec(),
    out_shape=jax.ShapeDtypeStruct((N, D), jnp.float32),
)

def kernel(x, edge_index, W_in, b_in, W_convs, W_out, b_out):
    src = edge_index[0].astype(jnp.int32)
    dst = edge_index[1].astype(jnp.int32)
    _deg_call, _edge_call = _sc_calls()

    deg_parts = _deg_call(dst)
    t0, u, dinv_col = _proj_call(deg_parts, x, W_in, b_in.reshape(1, D))

    dst3d = dst.reshape(NW, NCHUNK, CHUNK)
    p1 = _edge_call(t0, u, src, dst3d)
    t1 = _layer1_call(p1, p1, dinv_col, W_convs[0])

    p2 = _edge_call(t1, u, src, dst3d)
    y = _layer2_call(p2, p2, dinv_col, W_convs[1], W_out, b_out.reshape(1, D))
    return y
